# baseline (speedup 1.0000x reference)
import functools

import jax
import jax.numpy as jnp
from jax import lax
from jax.experimental import pallas as pl
from jax.experimental.pallas import tpu as pltpu
from jax.experimental.pallas import tpu_sc as plsc

NUM_E = 1_000_000
D = 64
NC = 2
NS = 16
L = 16
NW = NC * NS
CHUNK = 320
CW = CHUNK * D

TBR = 8192
NKS = 12
S_ROWS = NW * CHUNK * NKS
NCHUNK = S_ROWS // CHUNK
SOFF_B = S_ROWS // TBR
TC_GRID = -(-(NUM_E - S_ROWS) // TBR)

_NEG_INF = float("-inf")


def _scan_body(head_hbm, q_hbm, ent_hbm, vals_out, idx_out, rows_out,
               buf0, buf1, buf2, hbuf, qbuf, cv, ci, win, sem0, sem1, sem2):
    wid = lax.axis_index("s") * NC + lax.axis_index("c")
    nk = (NCHUNK - 1 - wid) // NW + 1

    pltpu.sync_copy(head_hbm, hbuf)
    pltpu.sync_copy(q_hbm, qbuf)

    h0 = hbuf[pl.ds(0, L)]
    h1 = hbuf[pl.ds(L, L)]
    h2 = hbuf[pl.ds(2 * L, L)]
    h3 = hbuf[pl.ds(3 * L, L)]
    q0 = qbuf[pl.ds(0, L)]
    q1 = qbuf[pl.ds(L, L)]
    q2 = qbuf[pl.ds(2 * L, L)]
    q3 = qbuf[pl.ds(3 * L, L)]
    wq0 = h0 * q0 - h2 * q2
    wq1 = h1 * q1 - h3 * q3
    wq2 = h2 * q0 + h0 * q2
    wq3 = h3 * q1 + h1 * q3

    lane = lax.broadcasted_iota(jnp.int32, (L,), 0)
    perms = [lane ^ k for k in (8, 4, 2, 1)]

    def hsum(v):
        for p in perms:
            v = v + v.at[p].get(mode="promise_in_bounds")
        return v[0]

    def off_of(k):
        return (wid + k * NW) * CW

    def start(buf, sem, k):
        pltpu.async_copy(ent_hbm.at[pl.ds(off_of(k), CW)], buf, sem)

    def wait(buf, sem, k):
        pltpu.make_async_copy(ent_hbm.at[pl.ds(off_of(k), CW)],
                              buf, sem).wait()

    def scan_chunk(buf, r0, carry):
        @plsc.parallel_loop(0, CHUNK, step=1, unroll=8, carry=carry)
        def row_body(r, c):
            bv, bi = c
            b = r * D
            t0 = buf[pl.ds(b, L)] * wq0
            t1 = buf[pl.ds(b + L, L)] * wq1
            t2 = buf[pl.ds(b + 2 * L, L)] * wq2
            t3 = buf[pl.ds(b + 3 * L, L)] * wq3
            s = hsum((t0 + t1) + (t2 + t3))
            ridx = r0 + r
            better = (s > bv) | ((s == bv) & (ridx < bi))
            bv = jnp.where(better, s, bv)
            bi = jnp.where(better, ridx, bi)
            return bv, bi
        return row_body

    bufs = (buf0, buf1, buf2)
    sems = (sem0, sem1, sem2)

    for s in range(3):
        start(bufs[s], sems[s], s)

    def outer(i3, carry):
        k0 = 3 * i3
        for s in range(3):
            buf, sem = bufs[s], sems[s]

            def do_stage(c, k=k0 + s, buf=buf, sem=sem):
                wait(buf, sem, k)
                c = scan_chunk(buf, (wid + k * NW) * CHUNK, c)

                @pl.when(k + 3 < nk)
                def _():
                    start(buf, sem, k + 3)

                return c

            carry = lax.cond(k0 + s < nk, do_stage, lambda c: c, carry)
        return carry

    n_outer = (nk + 2) // 3
    bv, bi = lax.fori_loop(0, n_outer, outer,
                           (jnp.float32(_NEG_INF), jnp.int32(0)))

    for i in range(8):
        cv[i] = jnp.full((L,), bv, jnp.float32)
        ci[i] = jnp.full((L,), bi, jnp.int32)
    pltpu.sync_copy(cv, vals_out.at[pl.ds(wid * 8, 8)])
    pltpu.sync_copy(ci, idx_out.at[pl.ds(wid * 8, 8)])
    woff = pl.multiple_of((bi // 2) * (2 * D), 2 * D)
    pltpu.sync_copy(ent_hbm.at[pl.ds(woff, 2 * D)], win)
    pltpu.sync_copy(win, rows_out.at[wid])


@functools.lru_cache(maxsize=None)
def _build_scan():
    mesh = plsc.VectorSubcoreMesh(core_axis_name="c", subcore_axis_name="s",
                                  num_cores=NC, num_subcores=NS)
    return pl.kernel(
        _scan_body,
        out_type=(
            jax.ShapeDtypeStruct((NW * 8, L), jnp.float32),
            jax.ShapeDtypeStruct((NW * 8, L), jnp.int32),
            jax.ShapeDtypeStruct((NW, 2 * D), jnp.float32),
        ),
        mesh=mesh,
        scratch_types=[
            pltpu.VMEM((CW,), jnp.float32),
            pltpu.VMEM((CW,), jnp.float32),
            pltpu.VMEM((CW,), jnp.float32),
            pltpu.VMEM((D,), jnp.float32),
            pltpu.VMEM((D,), jnp.float32),
            pltpu.VMEM((8, L), jnp.float32),
            pltpu.VMEM((8, L), jnp.int32),
            pltpu.VMEM((2 * D,), jnp.float32),
            pltpu.SemaphoreType.DMA,
            pltpu.SemaphoreType.DMA,
            pltpu.SemaphoreType.DMA,
        ],
    )


def _tc_body(head_ref, q_ref, blk_ref, ent_any, val_out, idx_out, win_out,
             sv, si, sem):
    pid = pl.program_id(0)

    @pl.when(pid == 0)
    def _():
        sv[0] = jnp.float32(_NEG_INF)
        si[0] = jnp.int32(0)

    h = head_ref[...]
    q = q_ref[...]
    a = h[:D // 2] * q[:D // 2] - h[D // 2:] * q[D // 2:]
    b = h[D // 2:] * q[:D // 2] + h[:D // 2] * q[D // 2:]
    w2 = jnp.concatenate([a, b, a, b])

    nv = TBR // 2
    prod = blk_ref[...].reshape(nv, 2 * D) * w2[None, :]
    low = jax.lax.broadcasted_iota(jnp.int32, (nv, 2 * D), 1) < D
    se = jnp.sum(jnp.where(low, prod, 0.0), axis=1)
    so = jnp.sum(jnp.where(low, 0.0, prod), axis=1)
    base = S_ROWS + pid * TBR
    gide = base + 2 * jax.lax.broadcasted_iota(jnp.int32, (nv,), 0)
    gido = gide + 1
    neg = jnp.float32(_NEG_INF)
    se = jnp.where(gide < NUM_E, se, neg)
    so = jnp.where(gido < NUM_E, so, neg)
    m = jnp.maximum(jnp.max(se), jnp.max(so))
    big = jnp.int32(jnp.iinfo(jnp.int32).max)
    lidx = jnp.minimum(jnp.min(jnp.where(se >= m, gide, big)),
                       jnp.min(jnp.where(so >= m, gido, big)))

    bv = sv[0]
    bi = si[0]
    better = (m > bv) | ((m == bv) & (lidx < bi))
    sv[0] = jnp.where(better, m, bv)
    si[0] = jnp.where(better, lidx, bi)

    @pl.when(pid == TC_GRID - 1)
    def _():
        val_out[...] = jnp.full((1,), sv[0], jnp.float32)
        idx_out[...] = jnp.full((1,), si[0], jnp.int32)
        woff = pl.multiple_of((si[0] // 2) * (2 * D), 2 * D)
        copy = pltpu.make_async_copy(ent_any.at[pl.ds(woff, 2 * D)],
                                     win_out, sem)
        copy.start()
        copy.wait()


_tc_scan = pl.pallas_call(
    _tc_body,
    grid=(TC_GRID,),
    out_shape=(
        jax.ShapeDtypeStruct((1,), jnp.float32),
        jax.ShapeDtypeStruct((1,), jnp.int32),
        jax.ShapeDtypeStruct((2 * D,), jnp.float32),
    ),
    in_specs=[
        pl.BlockSpec((D,), lambda i: (0,)),
        pl.BlockSpec((D,), lambda i: (0,)),
        pl.BlockSpec((TBR * D,), lambda i: (SOFF_B + i,)),
        pl.BlockSpec(memory_space=pltpu.MemorySpace.HBM),
    ],
    out_specs=(
        pl.BlockSpec((1,), lambda i: (0,)),
        pl.BlockSpec((1,), lambda i: (0,)),
        pl.BlockSpec((2 * D,), lambda i: (0,)),
    ),
    scratch_shapes=[
        pltpu.SMEM((1,), jnp.float32),
        pltpu.SMEM((1,), jnp.int32),
        pltpu.SemaphoreType.DMA,
    ],
)


def _merge_body(vals_ref, idx_ref, rows_ref, tval_ref, tidx_ref, twin_ref,
                out_ref):
    vals = vals_ref[...]
    idx = idx_ref[...]
    m = jnp.max(vals)
    big = jnp.int32(jnp.iinfo(jnp.int32).max)
    hit = vals >= m
    best = jnp.min(jnp.where(hit, idx, big))
    wids = jax.lax.broadcasted_iota(jnp.int32, (NW * 8, L), 0) // 8
    wstar = jnp.min(jnp.where(hit & (idx == best), wids, big))
    rows = rows_ref[...]
    rsel = jax.lax.broadcasted_iota(jnp.int32, (NW, 2 * D), 0) == wstar
    srow = jnp.sum(jnp.where(rsel, rows, 0.0), axis=0)
    row_sc = jnp.where(best % 2 == 0, srow[:D], srow[D:])

    tv = tval_ref[...][0]
    ti = tidx_ref[...][0]
    twin = twin_ref[...]
    row_tc = jnp.where(ti % 2 == 0, twin[:D], twin[D:])

    tc_wins = (tv > m) | ((tv == m) & (ti < best))
    out_ref[...] = jnp.where(tc_wins, row_tc, row_sc)


_merge = pl.pallas_call(
    _merge_body,
    out_shape=jax.ShapeDtypeStruct((D,), jnp.float32),
    in_specs=[pl.BlockSpec(memory_space=pltpu.VMEM)] * 6,
)


def kernel(head_entity, question_embedding, entity_embeddings):
    ent_flat = entity_embeddings.reshape(-1)
    ent_sc = jax.lax.slice(ent_flat, (0,), (S_ROWS * D,))
    vals, idx, rows = _build_scan()(head_entity, question_embedding,
                                    ent_sc)
    tval, tidx, twin = _tc_scan(head_entity, question_embedding,
                                ent_flat, ent_flat)
    return _merge(vals, idx, rows, tval, tidx, twin)


# speedup vs baseline: 1.5827x; 1.5827x over previous
import functools

import jax
import jax.numpy as jnp
from jax import lax
from jax.experimental import pallas as pl
from jax.experimental.pallas import tpu as pltpu
from jax.experimental.pallas import tpu_sc as plsc

NUM_E = 1_000_000
D = 64
NC = 2
NS = 16
L = 16
NW = NC * NS
CHUNK = 320
NCHUNK = NUM_E // CHUNK
WIN = 16


def _scan_body(head_hbm, q_hbm, ent_hbm, vals_out, idx_out, rows_out,
               buf0, buf1, buf2, hbuf, qbuf, cv, ci, win, sem0, sem1, sem2):
    wid = lax.axis_index("s") * NC + lax.axis_index("c")
    nk = (NCHUNK - 1 - wid) // NW + 1

    pltpu.sync_copy(head_hbm, hbuf)
    pltpu.sync_copy(q_hbm, qbuf)

    h0 = hbuf[pl.ds(0, L)]
    h1 = hbuf[pl.ds(L, L)]
    h2 = hbuf[pl.ds(2 * L, L)]
    h3 = hbuf[pl.ds(3 * L, L)]
    q0 = qbuf[pl.ds(0, L)]
    q1 = qbuf[pl.ds(L, L)]
    q2 = qbuf[pl.ds(2 * L, L)]
    q3 = qbuf[pl.ds(3 * L, L)]
    wq0 = h0 * q0 - h2 * q2
    wq1 = h1 * q1 - h3 * q3
    wq2 = h2 * q0 + h0 * q2
    wq3 = h3 * q1 + h1 * q3

    lane = lax.broadcasted_iota(jnp.int32, (L,), 0)
    perms = [lane ^ k for k in (8, 4, 2, 1)]

    def hsum(v):
        for p in perms:
            v = v + v.at[p].get(mode="promise_in_bounds")
        return v[0]

    def row0_of(k):
        return (wid + k * NW) * CHUNK

    def start(buf, sem, k):
        pltpu.async_copy(ent_hbm.at[pl.ds(row0_of(k), CHUNK)], buf, sem)

    def wait(buf, sem, k):
        pltpu.make_async_copy(ent_hbm.at[pl.ds(row0_of(k), CHUNK)],
                              buf, sem).wait()

    def scan_chunk(buf, r0, carry):
        @plsc.parallel_loop(0, CHUNK, step=1, unroll=8, carry=carry)
        def row_body(r, c):
            bv, bi = c
            t0 = buf[r, pl.ds(0, L)] * wq0
            t1 = buf[r, pl.ds(L, L)] * wq1
            t2 = buf[r, pl.ds(2 * L, L)] * wq2
            t3 = buf[r, pl.ds(3 * L, L)] * wq3
            s = hsum((t0 + t1) + (t2 + t3))
            ridx = r0 + r
            better = (s > bv) | ((s == bv) & (ridx < bi))
            bv = jnp.where(better, s, bv)
            bi = jnp.where(better, ridx, bi)
            return bv, bi
        return row_body

    bufs = (buf0, buf1, buf2)
    sems = (sem0, sem1, sem2)

    for s in range(3):
        start(bufs[s], sems[s], s)

    def outer(i3, carry):
        k0 = 3 * i3
        for s in range(3):
            buf, sem = bufs[s], sems[s]

            def do_stage(c, k=k0 + s, buf=buf, sem=sem):
                wait(buf, sem, k)
                c = scan_chunk(buf, row0_of(k), c)

                @pl.when(k + 3 < nk)
                def _():
                    start(buf, sem, k + 3)

                return c

            carry = lax.cond(k0 + s < nk, do_stage, lambda c: c, carry)
        return carry

    n_outer = (nk + 2) // 3
    bv, bi = lax.fori_loop(0, n_outer, outer,
                           (jnp.float32(-jnp.inf), jnp.int32(0)))

    for i in range(8):
        cv[i] = jnp.full((L,), bv, jnp.float32)
        ci[i] = jnp.full((L,), bi, jnp.int32)
    pltpu.sync_copy(cv, vals_out.at[pl.ds(wid * 8, 8)])
    pltpu.sync_copy(ci, idx_out.at[pl.ds(wid * 8, 8)])
    wrow = pl.multiple_of((bi // WIN) * WIN, WIN)
    pltpu.sync_copy(ent_hbm.at[pl.ds(wrow, WIN)], win)
    pltpu.sync_copy(win, rows_out.at[pl.ds(wid * WIN, WIN)])


@functools.lru_cache(maxsize=None)
def _build_scan():
    mesh = plsc.VectorSubcoreMesh(core_axis_name="c", subcore_axis_name="s",
                                  num_cores=NC, num_subcores=NS)
    return pl.kernel(
        _scan_body,
        out_type=(
            jax.ShapeDtypeStruct((NW * 8, L), jnp.float32),
            jax.ShapeDtypeStruct((NW * 8, L), jnp.int32),
            jax.ShapeDtypeStruct((NW * WIN, D), jnp.float32),
        ),
        mesh=mesh,
        scratch_types=[
            pltpu.VMEM((CHUNK, D), jnp.float32),
            pltpu.VMEM((CHUNK, D), jnp.float32),
            pltpu.VMEM((CHUNK, D), jnp.float32),
            pltpu.VMEM((D,), jnp.float32),
            pltpu.VMEM((D,), jnp.float32),
            pltpu.VMEM((8, L), jnp.float32),
            pltpu.VMEM((8, L), jnp.int32),
            pltpu.VMEM((WIN, D), jnp.float32),
            pltpu.SemaphoreType.DMA,
            pltpu.SemaphoreType.DMA,
            pltpu.SemaphoreType.DMA,
        ],
    )


def _merge_body(vals_ref, idx_ref, rows_ref, out_ref):
    vals = vals_ref[...]
    idx = idx_ref[...]
    m = jnp.max(vals)
    big = jnp.int32(jnp.iinfo(jnp.int32).max)
    hit = vals >= m
    best = jnp.min(jnp.where(hit, idx, big))
    wids = jax.lax.broadcasted_iota(jnp.int32, (NW * 8, L), 0) // 8
    wstar = jnp.min(jnp.where(hit & (idx == best), wids, big))
    rows = rows_ref[...]
    gsel = wstar * WIN + best % WIN
    rsel = jax.lax.broadcasted_iota(jnp.int32, (NW * WIN, D), 0) == gsel
    out_ref[...] = jnp.sum(jnp.where(rsel, rows, 0.0), axis=0)


_merge = pl.pallas_call(
    _merge_body,
    out_shape=jax.ShapeDtypeStruct((D,), jnp.float32),
    in_specs=[
        pl.BlockSpec(memory_space=pltpu.VMEM),
        pl.BlockSpec(memory_space=pltpu.VMEM),
        pl.BlockSpec(memory_space=pltpu.VMEM),
    ],
)


def kernel(head_entity, question_embedding, entity_embeddings):
    vals, idx, rows = _build_scan()(head_entity, question_embedding,
                                    entity_embeddings)
    return _merge(vals, idx, rows)
